# Initial kernel scaffold; baseline (speedup 1.0000x reference)
import jax
import jax.numpy as jnp
from jax.experimental import pallas as pl


def kernel(logits, correctness):
    raise NotImplementedError("write your pallas kernel here")


# speedup vs baseline: 2.4539x; 2.4539x over previous
import functools

import jax
import jax.numpy as jnp
from jax import lax
from jax.experimental import pallas as pl
from jax.experimental.pallas import tpu as pltpu
from jax.experimental.pallas import tpu_sc as plsc

_N = 4194304
_NBINS = 10
_NC = 2
_NS = 16
_NW = _NC * _NS
_PER_W = _N // _NW
_CHUNK = 16384
_NBUF = 3
_NCHUNK = _PER_W // _CHUNK
_VPB = 16
_L = 16
_ROWS = _NBINS

_CMUL = 10.0 * (1.0 - 2.0 ** -23)
_mesh = plsc.VectorSubcoreMesh(core_axis_name="c", subcore_axis_name="s")


@functools.partial(
    pl.kernel,
    mesh=_mesh,
    compiler_params=pltpu.CompilerParams(needs_layout_passes=False),
    out_type=[
        jax.ShapeDtypeStruct((_NW * _ROWS * _L,), jnp.float32),
        jax.ShapeDtypeStruct((_NW * _ROWS * _L,), jnp.int32),
    ],
    scratch_types=[
        pltpu.VMEM((_NBUF * _CHUNK,), jnp.float32),
        pltpu.VMEM((_NBUF * _CHUNK,), jnp.int32),
        pltpu.VMEM((_ROWS * _L,), jnp.float32),
        pltpu.VMEM((_ROWS * _L,), jnp.int32),
        pltpu.SemaphoreType.DMA,
        pltpu.SemaphoreType.DMA,
        pltpu.SemaphoreType.DMA,
        pltpu.SemaphoreType.DMA,
        pltpu.SemaphoreType.DMA,
        pltpu.SemaphoreType.DMA,
    ],
)
def _ece_partials(logits_hbm, corr_hbm, conf_out, ca_out,
                  lbuf, cbuf, conf_acc, ca_acc,
                  sl0, sl1, sl2, sc0, sc1, sc2):
    sls = (sl0, sl1, sl2)
    scs = (sc0, sc1, sc2)
    wid = lax.axis_index("c") * _NS + lax.axis_index("s")
    base = wid * _PER_W

    zeros = jnp.zeros((_L,), jnp.float32)
    izeros = jnp.zeros((_L,), jnp.int32)
    for j in range(_ROWS):
        conf_acc[pl.ds(j * _L, _L)] = zeros
        ca_acc[pl.ds(j * _L, _L)] = izeros

    lane = lax.iota(jnp.int32, _L)

    def start(g):
        s = g % _NBUF
        pltpu.async_copy(
            logits_hbm.at[pl.ds(base + g * _CHUNK, _CHUNK)],
            lbuf.at[pl.ds(s * _CHUNK, _CHUNK)], sls[s])
        pltpu.async_copy(
            corr_hbm.at[pl.ds(base + g * _CHUNK, _CHUNK)],
            cbuf.at[pl.ds(s * _CHUNK, _CHUNK)], scs[s])

    def wait(g):
        s = g % _NBUF
        pltpu.make_async_copy(
            logits_hbm.at[pl.ds(base + g * _CHUNK, _CHUNK)],
            lbuf.at[pl.ds(s * _CHUNK, _CHUNK)], sls[s]).wait()
        pltpu.make_async_copy(
            corr_hbm.at[pl.ds(base + g * _CHUNK, _CHUNK)],
            cbuf.at[pl.ds(s * _CHUNK, _CHUNK)], scs[s]).wait()

    def compute(g):
        s = g % _NBUF
        sbase = s * _CHUNK

        def body(i, carry):
            off = sbase + i * (_L * _VPB)
            xs = [lbuf[pl.ds(off + u * _L, _L)] for u in range(_VPB)]
            cs = [cbuf[pl.ds(off + u * _L, _L)] for u in range(_VPB)]
            for u in range(_VPB):
                x, c = xs[u], cs[u]
                ki = (x * jnp.float32(_CMUL)).astype(jnp.int32)
                idx = ki * _L + lane
                packed = c + jnp.int32(16384)
                pos = x > jnp.float32(0.0)
                plsc.addupdate_scatter(conf_acc, [idx], x, mask=pos)
                plsc.addupdate_scatter(ca_acc, [idx], packed, mask=pos)
            return carry

        lax.fori_loop(0, _CHUNK // (_L * _VPB), body, 0)

    for g in range(min(_NBUF, _NCHUNK)):
        start(g)
    for g in range(_NCHUNK):
        wait(g)
        compute(g)
        if g + _NBUF < _NCHUNK:
            start(g + _NBUF)

    obase = wid * (_ROWS * _L)
    pltpu.sync_copy(conf_acc, conf_out.at[pl.ds(obase, _ROWS * _L)])
    pltpu.sync_copy(ca_acc, ca_out.at[pl.ds(obase, _ROWS * _L)])


def kernel(logits, correctness):
    conf_p, ca_p = _ece_partials(logits, correctness.astype(jnp.int32))
    ca = ca_p.reshape(_NW, _ROWS, _L)
    count = (ca >> 14).sum(axis=(0, 2)).astype(jnp.float32)
    sum_acc = (ca & 16383).sum(axis=(0, 2)).astype(jnp.float32)
    sum_conf = conf_p.reshape(_NW, _ROWS, _L).sum(axis=(0, 2))
    total = jnp.float32(logits.size)
    safe = jnp.maximum(count, 1.0)
    contrib = jnp.abs(sum_conf / safe - sum_acc / safe) * (count / total)
    ece = jnp.sum(jnp.where(count > 0, contrib, 0.0))
    return ece.reshape(1)
